# baseline (speedup 1.0000x reference)
import functools

import jax
import jax.numpy as jnp
from jax import lax
from jax.experimental import pallas as pl
from jax.experimental.pallas import tpu as pltpu
from jax.experimental.pallas import tpu_sc as plsc

BB = 16
HALF = 100
HPAD = 128


def _rank_block(ts_ref, emb_ref, idx0_ref, idx1_ref, *, dump_row):
    t = ts_ref[...]
    bb, hist = t.shape
    tk = t[:, :, None]
    tj = t[:, None, :]
    kk2 = lax.broadcasted_iota(jnp.int32, (hist, hist), 0)
    jj2 = lax.broadcasted_iota(jnp.int32, (hist, hist), 1)
    tri = (kk2 < jj2)[None]
    c = ((tk < tj) | ((tk <= tj) & tri)).astype(jnp.int32)
    rank = jnp.sum(c, axis=1)
    row = pl.program_id(0) * bb + lax.broadcasted_iota(jnp.int32, (bb, 1), 0)
    gidx = rank + row * hist
    pad = jnp.full((bb, HPAD - HALF), dump_row, jnp.int32)
    idx0_ref[...] = jnp.concatenate([gidx[:, :HALF], pad], axis=1)
    idx1_ref[...] = jnp.concatenate([gidx[:, HALF:], pad], axis=1)


def _sc_scatter(idx0_hbm, idx1_hbm, emb_hbm, out_hbm,
                table_v, idx0_v, idx1_v, sem, *, rows_per_w, hist, expand):
    nc = plsc.get_sparse_core_info().num_cores
    wid = lax.axis_index("s") * nc + lax.axis_index("c")
    base = wid * rows_per_w
    pltpu.sync_copy(emb_hbm, table_v.at[pl.ds(0, hist)])
    src0 = table_v.at[pl.ds(0, HPAD)]
    src1 = table_v.at[pl.ds(HALF, HPAD)]
    ch = idx0_v.shape[0]

    def chunk_body(cn, _):
        cb = base + cn * ch
        pltpu.sync_copy(idx0_hbm.at[pl.ds(cb, ch)], idx0_v)
        pltpu.sync_copy(idx1_hbm.at[pl.ds(cb, ch)], idx1_v)

        def fire(r, _):
            pltpu.make_async_copy(src0, out_hbm.at[idx0_v.at[r]], sem).start()
            pltpu.make_async_copy(src1, out_hbm.at[idx1_v.at[r]], sem).start()
            return ()

        lax.fori_loop(0, ch, fire, (), unroll=4)

        def drain(r, _):
            pltpu.make_async_copy(src0, out_hbm.at[idx0_v.at[r]], sem).wait()
            pltpu.make_async_copy(src1, out_hbm.at[idx1_v.at[r]], sem).wait()
            return ()

        lax.fori_loop(0, ch, drain, (), unroll=4)
        return ()

    lax.fori_loop(0, rows_per_w // ch, chunk_body, ())


def kernel(ts, pos_embeddings):
    batch, hist = ts.shape
    seq_len, expand = pos_embeddings.shape
    nrows = batch * hist
    dump_row = nrows

    idx0, idx1 = pl.pallas_call(
        functools.partial(_rank_block, dump_row=dump_row),
        grid=(batch // BB,),
        in_specs=[
            pl.BlockSpec((BB, hist), lambda i: (i, 0)),
            pl.BlockSpec((seq_len, expand), lambda i: (0, 0)),
        ],
        out_specs=[
            pl.BlockSpec((BB, HPAD), lambda i: (i, 0)),
            pl.BlockSpec((BB, HPAD), lambda i: (i, 0)),
        ],
        out_shape=[
            jax.ShapeDtypeStruct((batch, HPAD), jnp.int32),
            jax.ShapeDtypeStruct((batch, HPAD), jnp.int32),
        ],
    )(ts, pos_embeddings)

    info = plsc.get_sparse_core_info()
    nw = info.num_cores * info.num_subcores
    rows_per_w = batch // nw
    mesh = plsc.VectorSubcoreMesh(core_axis_name="c", subcore_axis_name="s")
    scatter = pl.kernel(
        functools.partial(_sc_scatter, rows_per_w=rows_per_w,
                          hist=hist, expand=expand),
        mesh=mesh,
        compiler_params=pltpu.CompilerParams(use_tc_tiling_on_sc=False),
        out_type=jax.ShapeDtypeStruct((nrows + 8, expand), jnp.float32),
        scratch_types=[
            pltpu.VMEM((hist + HPAD - HALF, expand), jnp.float32),
            pltpu.VMEM((128, HPAD), jnp.int32),
            pltpu.VMEM((128, HPAD), jnp.int32),
            pltpu.SemaphoreType.DMA,
        ],
    )
    flat = scatter(idx0, idx1, pos_embeddings)
    return flat[:nrows].reshape(batch, hist, expand)


# speedup vs baseline: 3.2880x; 3.2880x over previous
import jax
import jax.numpy as jnp
from jax import lax
from jax.experimental import pallas as pl

BB = 16


def _posenc_block(ts_ref, emb_ref, out_ref):
    t = ts_ref[...]
    bb, hist = t.shape
    expand = emb_ref.shape[1]
    tk = t[:, :, None]
    tj = t[:, None, :]
    kk2 = lax.broadcasted_iota(jnp.int32, (hist, hist), 0)
    jj2 = lax.broadcasted_iota(jnp.int32, (hist, hist), 1)
    tri = (kk2 < jj2)[None]
    c = ((tk < tj) | ((tk <= tj) & tri)).astype(jnp.int32)
    rank = jnp.sum(c, axis=1)
    hq = hist // 4
    e = emb_ref[...]
    rank_b = jnp.repeat(rank, hq, axis=0)
    ih4 = (lax.broadcasted_iota(jnp.int32, (bb * hq, 1), 0) % hq) * 4
    outs = []
    for il in range(4):
        m_il = (rank_b == ih4 + il).astype(jnp.float32)
        outs.append(jnp.dot(m_il, e, preferred_element_type=jnp.float32))
    out_ref[...] = jnp.concatenate(outs, axis=1)


def kernel(ts, pos_embeddings):
    batch, hist = ts.shape
    seq_len, expand = pos_embeddings.shape
    hq = hist // 4
    flat = pl.pallas_call(
        _posenc_block,
        grid=(batch // BB,),
        in_specs=[
            pl.BlockSpec((BB, hist), lambda i: (i, 0)),
            pl.BlockSpec((seq_len, expand), lambda i: (0, 0)),
        ],
        out_specs=pl.BlockSpec((BB * hq, 4 * expand), lambda i: (i, 0)),
        out_shape=jax.ShapeDtypeStruct((batch * hq, 4 * expand),
                                       jnp.float32),
    )(ts, pos_embeddings)
    return flat.reshape(batch, hist, expand)


# speedup vs baseline: 5.9381x; 1.8060x over previous
import jax
import jax.numpy as jnp
from jax import lax
from jax.experimental import pallas as pl

BB = 32


def _posenc_block(ts_ref, emb_ref, out_ref):
    t = ts_ref[...]
    bb, hist = t.shape
    expand = emb_ref.shape[1]
    tk = t[:, :, None]
    tj = t[:, None, :]
    kk2 = lax.broadcasted_iota(jnp.int32, (hist, hist), 0)
    jj2 = lax.broadcasted_iota(jnp.int32, (hist, hist), 1)
    tri = (kk2 < jj2)[None]
    c = ((tk < tj) | ((tk <= tj) & tri)).astype(jnp.int32)
    rank = jnp.sum(c, axis=1)
    ii = lax.broadcasted_iota(jnp.int32, (bb, hist, hist), 1)
    m = (rank[:, None, :] == ii).astype(jnp.float32)
    out = jnp.dot(m.reshape(bb * hist, hist), emb_ref[...],
                  preferred_element_type=jnp.float32)
    out_ref[...] = out.reshape(bb, hist, expand)


def kernel(ts, pos_embeddings):
    batch, hist = ts.shape
    seq_len, expand = pos_embeddings.shape
    return pl.pallas_call(
        _posenc_block,
        grid=(batch // BB,),
        in_specs=[
            pl.BlockSpec((BB, hist), lambda i: (i, 0)),
            pl.BlockSpec((seq_len, expand), lambda i: (0, 0)),
        ],
        out_specs=pl.BlockSpec((BB, hist, expand), lambda i: (i, 0, 0)),
        out_shape=jax.ShapeDtypeStruct((batch, hist, expand), jnp.float32),
    )(ts, pos_embeddings)


# speedup vs baseline: 6.1117x; 1.0292x over previous
import jax
import jax.numpy as jnp
from jax import lax
from jax.experimental import pallas as pl

BB = 64


def _posenc_block(ts_ref, emb_ref, out_ref):
    t = ts_ref[...]
    bb, hist = t.shape
    expand = emb_ref.shape[1]
    tk = t[:, :, None]
    tj = t[:, None, :]
    kk2 = lax.broadcasted_iota(jnp.int32, (hist, hist), 0)
    jj2 = lax.broadcasted_iota(jnp.int32, (hist, hist), 1)
    tri = (kk2 < jj2)[None]
    c = ((tk < tj) | ((tk <= tj) & tri)).astype(jnp.int32)
    rank = jnp.sum(c, axis=1)
    ii = lax.broadcasted_iota(jnp.int32, (bb, hist, hist), 1)
    m = (rank[:, None, :] == ii).astype(jnp.float32)
    out = jnp.dot(m.reshape(bb * hist, hist), emb_ref[...],
                  preferred_element_type=jnp.float32)
    out_ref[...] = out.reshape(bb, hist, expand)


def kernel(ts, pos_embeddings):
    batch, hist = ts.shape
    seq_len, expand = pos_embeddings.shape
    return pl.pallas_call(
        _posenc_block,
        grid=(batch // BB,),
        in_specs=[
            pl.BlockSpec((BB, hist), lambda i: (i, 0)),
            pl.BlockSpec((seq_len, expand), lambda i: (0, 0)),
        ],
        out_specs=pl.BlockSpec((BB, hist, expand), lambda i: (i, 0, 0)),
        out_shape=jax.ShapeDtypeStruct((batch, hist, expand), jnp.float32),
    )(ts, pos_embeddings)


# speedup vs baseline: 6.1801x; 1.0112x over previous
import jax
import jax.numpy as jnp
from jax import lax
from jax.experimental import pallas as pl

BB = 128


def _posenc_block(ts_ref, emb_ref, out_ref):
    t = ts_ref[...]
    bb, hist = t.shape
    expand = emb_ref.shape[1]
    tk = t[:, :, None]
    tj = t[:, None, :]
    kk2 = lax.broadcasted_iota(jnp.int32, (hist, hist), 0)
    jj2 = lax.broadcasted_iota(jnp.int32, (hist, hist), 1)
    tri = (kk2 < jj2)[None]
    c = ((tk < tj) | ((tk <= tj) & tri)).astype(jnp.int32)
    rank = jnp.sum(c, axis=1)
    ii = lax.broadcasted_iota(jnp.int32, (bb, hist, hist), 1)
    m = (rank[:, None, :] == ii).astype(jnp.float32)
    out = jnp.dot(m.reshape(bb * hist, hist), emb_ref[...],
                  preferred_element_type=jnp.float32)
    out_ref[...] = out.reshape(bb, hist, expand)


def kernel(ts, pos_embeddings):
    batch, hist = ts.shape
    seq_len, expand = pos_embeddings.shape
    return pl.pallas_call(
        _posenc_block,
        grid=(batch // BB,),
        in_specs=[
            pl.BlockSpec((BB, hist), lambda i: (i, 0)),
            pl.BlockSpec((seq_len, expand), lambda i: (0, 0)),
        ],
        out_specs=pl.BlockSpec((BB, hist, expand), lambda i: (i, 0, 0)),
        out_shape=jax.ShapeDtypeStruct((batch, hist, expand), jnp.float32),
    )(ts, pos_embeddings)
